# baseline (speedup 1.0000x reference)
import functools
import math

import jax
import jax.numpy as jnp
from jax import lax
from jax.experimental import pallas as pl
from jax.experimental.pallas import tpu as pltpu
from jax.experimental.pallas import tpu_sc as plsc

_B = 64
_S = 512
_N = _B * _S
_D4 = 32
_D = 128

_NC = 2
_NS = 16
_NW = _NC * _NS
_B_PER_W = _N // _NW
_CHUNK = 128
_NCHUNK = _B_PER_W // _CHUNK

_SPB = 8


_PAD_ROWS = 5000


def _pad_body(t_ref, out_ref):
    out_ref[:, :_D4] = t_ref[...]
    out_ref[:, _D4:] = jnp.zeros((_PAD_ROWS, _D - _D4), jnp.float32)


def _tc_pad(token_table):
    v = token_table.shape[0]
    return pl.pallas_call(
        _pad_body,
        grid=(v // _PAD_ROWS,),
        in_specs=[pl.BlockSpec((_PAD_ROWS, _D4), lambda i: (i, 0))],
        out_specs=pl.BlockSpec((_PAD_ROWS, _D), lambda i: (i, 0)),
        out_shape=jax.ShapeDtypeStruct((v, _D), jnp.float32),
    )(token_table)


def _sc_gather_kernel(table_hbm, idx_hbm, out_hbm, idx_v, buf_v, sem):
    wid = lax.axis_index("s") * _NC + lax.axis_index("c")
    base = wid * _B_PER_W
    pltpu.sync_copy(idx_hbm.at[pl.ds(base, _B_PER_W)], idx_v)
    copies = []
    for j in range(_NCHUNK):
        copies.append(
            pltpu.async_copy(
                table_hbm.at[idx_v.at[pl.ds(j * _CHUNK, _CHUNK)]],
                buf_v.at[j % 2],
                sem,
            )
        )
        if j > 0:
            copies[j - 1].wait()
            pltpu.sync_copy(
                buf_v.at[(j - 1) % 2],
                out_hbm.at[pl.ds(base + (j - 1) * _CHUNK, _CHUNK)],
            )
    copies[_NCHUNK - 1].wait()
    pltpu.sync_copy(
        buf_v.at[(_NCHUNK - 1) % 2],
        out_hbm.at[pl.ds(base + (_NCHUNK - 1) * _CHUNK, _CHUNK)],
    )


def _sc_gather(tablep, idx_flat):
    mesh = plsc.VectorSubcoreMesh(core_axis_name="c", subcore_axis_name="s")
    k = pl.kernel(
        _sc_gather_kernel,
        out_type=jax.ShapeDtypeStruct((_N, _D), jnp.float32),
        mesh=mesh,
        compiler_params=pltpu.CompilerParams(use_tc_tiling_on_sc=True),
        scratch_types=[
            pltpu.VMEM((_B_PER_W,), jnp.int32),
            pltpu.VMEM((2, _CHUNK, _D), jnp.float32),
            pltpu.SemaphoreType.DMA,
        ],
    )
    return k(tablep, idx_flat)


def _tc_body(g_ref, tt_ref, w_ref, b_ref, pos_ref, type_ref, gm_ref, bt_ref,
             out_ref):
    g = g_ref[...]
    lane = lax.broadcasted_iota(jnp.int32, (_SPB * _S, _D), 1)
    h = 0.5 * g * (1.0 + lax.erf(g * (1.0 / math.sqrt(2.0))))
    h = jnp.where(lane < _D4, h, 0.0)
    w = w_ref[...]
    wpad = jnp.concatenate([w, jnp.zeros((_D - _D4, _D), jnp.float32)], axis=0)
    h2 = jnp.dot(h, wpad, preferred_element_type=jnp.float32)
    h3 = h2.reshape(_SPB, _S, _D)
    h3 = h3 + b_ref[...].reshape(1, 1, _D)
    h3 = h3 + pos_ref[...][None, :, :]
    ty0 = type_ref[0, :].reshape(1, 1, _D)
    tyd = (type_ref[1, :] - type_ref[0, :]).reshape(1, 1, _D)
    tt = tt_ref[...].astype(jnp.float32)
    h3 = h3 + ty0 + tt[:, :, None] * tyd
    mean = jnp.mean(h3, axis=-1, keepdims=True)
    d = h3 - mean
    var = jnp.mean(d * d, axis=-1, keepdims=True)
    out_ref[...] = (d * lax.rsqrt(var + 1e-12)) * gm_ref[...].reshape(1, 1, _D) \
        + bt_ref[...].reshape(1, 1, _D)


def _tc_compute(gathered, token_type, proj_W, proj_b, pos, type_table, gamma,
                beta):
    grid = (_B // _SPB,)
    full = lambda i: (0, 0)
    return pl.pallas_call(
        _tc_body,
        grid=grid,
        in_specs=[
            pl.BlockSpec((_SPB * _S, _D), lambda i: (i, 0)),
            pl.BlockSpec((_SPB, _S), lambda i: (i, 0)),
            pl.BlockSpec((_D4, _D), full),
            pl.BlockSpec((1, _D), full),
            pl.BlockSpec((_S, _D), full),
            pl.BlockSpec((2, _D), full),
            pl.BlockSpec((1, _D), full),
            pl.BlockSpec((1, _D), full),
        ],
        out_specs=pl.BlockSpec((_SPB, _S, _D), lambda i: (i, 0, 0)),
        out_shape=jax.ShapeDtypeStruct((_B, _S, _D), jnp.float32),
    )(gathered, token_type, proj_W, proj_b, pos, type_table, gamma, beta)


def kernel(x, token_type, token_table, proj_W, proj_b, pos_table, type_table,
           gamma, beta):
    tablep = _tc_pad(token_table)
    idx_flat = x.reshape(_N)
    gathered = _sc_gather(tablep, idx_flat)
    return _tc_compute(
        gathered,
        token_type,
        proj_W,
        proj_b.reshape(1, _D),
        pos_table[:_S],
        type_table,
        gamma.reshape(1, _D),
        beta.reshape(1, _D),
    )


# speedup vs baseline: 1.2094x; 1.2094x over previous
import functools
import math

import jax
import jax.numpy as jnp
from jax import lax
from jax.experimental import pallas as pl
from jax.experimental.pallas import tpu as pltpu
from jax.experimental.pallas import tpu_sc as plsc

_B = 64
_S = 512
_N = _B * _S
_D4 = 32
_D = 128
_PACK = _D // _D4
_NP = _N // _PACK
_Q = _S // _PACK

_NC = 2
_NS = 16
_NW = _NC * _NS
_B_PER_W = _N // _NW
_CHUNK = 128
_NCHUNK = _B_PER_W // _CHUNK

_RPB = 1024
_SPB = _RPB // _Q


def _sc_gather_kernel(table_hbm, idx_hbm, out_hbm, idx_v, rows_v, sem):
    wid = lax.axis_index("s") * _NC + lax.axis_index("c")
    pltpu.sync_copy(idx_hbm.at[pl.ds(2 * wid, 2)], idx_v)
    copies = []
    for j in range(_NCHUNK):
        copies.append(
            pltpu.async_copy(
                table_hbm.at[idx_v.at[j // _PACK,
                                      pl.ds((j % _PACK) * _CHUNK, _CHUNK)]],
                rows_v.at[pl.ds(j * _CHUNK, _CHUNK)],
                sem,
            )
        )
    for c in copies:
        c.wait()
    for j in range(_NCHUNK):
        s = 2 * wid + (1 if j >= _PACK else 0)
        k = j % _PACK
        pltpu.sync_copy(
            rows_v.at[pl.ds(j * _CHUNK, _CHUNK)],
            out_hbm.at[pl.ds(s * _Q, _Q), pl.ds(k * _D4, _D4)],
        )


def _sc_gather(token_table, x):
    mesh = plsc.VectorSubcoreMesh(core_axis_name="c", subcore_axis_name="s")
    k = pl.kernel(
        _sc_gather_kernel,
        out_type=jax.ShapeDtypeStruct((_NP, _D), jnp.float32),
        mesh=mesh,
        compiler_params=pltpu.CompilerParams(use_tc_tiling_on_sc=False),
        scratch_types=[
            pltpu.VMEM((2, _S), jnp.int32),
            pltpu.VMEM((_B_PER_W, _D4), jnp.float32),
            pltpu.SemaphoreType.DMA,
        ],
    )
    return k(token_table, x)


def _tc_body(g_ref, tt_ref, w_ref, b_ref, pos_ref, type_ref, gm_ref, bt_ref,
             out_ref):
    g = g_ref[...]
    h = 0.5 * g * (1.0 + lax.erf(g * (1.0 / math.sqrt(2.0))))
    w = w_ref[...]
    c4 = jnp.concatenate([w, w, w, w], axis=0)
    wrep = jnp.concatenate([c4, c4, c4, c4], axis=1)
    rowq = lax.broadcasted_iota(jnp.int32, (_D, _PACK * _D), 0) // _D4
    colq = lax.broadcasted_iota(jnp.int32, (_D, _PACK * _D), 1) // _D
    w512 = jnp.where(rowq == colq, wrep, 0.0)
    h2 = jnp.dot(h, w512, preferred_element_type=jnp.float32)

    bias = b_ref[...].reshape(1, 1, _D)
    ty0 = type_ref[0, :].reshape(1, 1, _D)
    tyd = (type_ref[1, :] - type_ref[0, :]).reshape(1, 1, _D)
    gm = gm_ref[...].reshape(1, 1, _D)
    bt = bt_ref[...].reshape(1, 1, _D)
    tb = tt_ref[...].astype(jnp.int32)
    pieces = []
    for k in range(_PACK):
        s = h2[:, k * _D:(k + 1) * _D]
        s = s.reshape(_SPB, _Q, _D)
        bitk = ((tb >> k) & 1).astype(jnp.float32).reshape(_SPB, _Q, 1)
        s = s + bias + pos_ref[k * _Q:(k + 1) * _Q, :][None, :, :] \
            + ty0 + bitk * tyd
        mean = jnp.mean(s, axis=-1, keepdims=True)
        d = s - mean
        var = jnp.mean(d * d, axis=-1, keepdims=True)
        pieces.append((d * lax.rsqrt(var + 1e-12)) * gm + bt)
    out_ref[...] = jnp.stack(pieces, axis=1)


def _tc_compute(gathered, ttp, proj_W, proj_b, pos, type_table, gamma, beta):
    grid = (_NP // _RPB,)
    full = lambda i: (0, 0)
    return pl.pallas_call(
        _tc_body,
        grid=grid,
        in_specs=[
            pl.BlockSpec((_RPB, _D), lambda i: (i, 0)),
            pl.BlockSpec((_RPB, 1), lambda i: (i, 0)),
            pl.BlockSpec((_D4, _D), full),
            pl.BlockSpec((1, _D), full),
            pl.BlockSpec((_S, _D), full),
            pl.BlockSpec((2, _D), full),
            pl.BlockSpec((1, _D), full),
            pl.BlockSpec((1, _D), full),
        ],
        out_specs=pl.BlockSpec((_SPB, _PACK, _Q, _D), lambda i: (i, 0, 0, 0)),
        out_shape=jax.ShapeDtypeStruct((_B, _PACK, _Q, _D), jnp.float32),
    )(gathered, ttp, proj_W, proj_b, pos, type_table, gamma, beta)


def kernel(x, token_type, token_table, proj_W, proj_b, pos_table, type_table,
           gamma, beta):
    gathered = _sc_gather(token_table, x)
    ttq = token_type.reshape(_B, _PACK, _Q)
    ttp = (ttq[:, 0] | (ttq[:, 1] << 1) | (ttq[:, 2] << 2)
           | (ttq[:, 3] << 3)).astype(jnp.int8).reshape(_NP, 1)
    out = _tc_compute(
        gathered,
        ttp,
        proj_W,
        proj_b.reshape(1, _D),
        pos_table[:_S],
        type_table,
        gamma.reshape(1, _D),
        beta.reshape(1, _D),
    )
    return out.reshape(_B, _S, _D)


# speedup vs baseline: 1.2137x; 1.0036x over previous
import functools
import math

import jax
import jax.numpy as jnp
from jax import lax
from jax.experimental import pallas as pl
from jax.experimental.pallas import tpu as pltpu
from jax.experimental.pallas import tpu_sc as plsc

_B = 64
_S = 512
_N = _B * _S
_D4 = 32
_D = 128
_PACK = _D // _D4
_NP = _N // _PACK
_Q = _S // _PACK

_NC = 2
_NS = 16
_NW = _NC * _NS
_B_PER_W = _N // _NW
_CHUNK = 128
_NCHUNK = _B_PER_W // _CHUNK

_RPB = 1024
_SPB = _RPB // _Q


def _sc_gather_kernel(table_hbm, idx_hbm, out_hbm, idx_v, rows_v, sem):
    wid = lax.axis_index("s") * _NC + lax.axis_index("c")
    pltpu.sync_copy(idx_hbm.at[pl.ds(2 * wid, 2)], idx_v)
    copies = []
    for j in range(_NCHUNK):
        copies.append(
            pltpu.async_copy(
                table_hbm.at[idx_v.at[j // _PACK,
                                      pl.ds((j % _PACK) * _CHUNK, _CHUNK)]],
                rows_v.at[pl.ds(j * _CHUNK, _CHUNK)],
                sem,
            )
        )
    for c in copies:
        c.wait()
    for j in range(_NCHUNK):
        s = 2 * wid + (1 if j >= _PACK else 0)
        k = j % _PACK
        pltpu.sync_copy(
            rows_v.at[pl.ds(j * _CHUNK, _CHUNK)],
            out_hbm.at[pl.ds(s * _Q, _Q), pl.ds(k * _D4, _D4)],
        )


def _sc_gather(token_table, x):
    mesh = plsc.VectorSubcoreMesh(core_axis_name="c", subcore_axis_name="s")
    k = pl.kernel(
        _sc_gather_kernel,
        out_type=jax.ShapeDtypeStruct((_NP, _D), jnp.float32),
        mesh=mesh,
        compiler_params=pltpu.CompilerParams(use_tc_tiling_on_sc=False),
        scratch_types=[
            pltpu.VMEM((2, _S), jnp.int32),
            pltpu.VMEM((_B_PER_W, _D4), jnp.float32),
            pltpu.SemaphoreType.DMA,
        ],
    )
    return k(token_table, x)


def _tc_body(g_ref, tt_ref, w_ref, posb_ref, tyd_ref, gm_ref, bt_ref,
             out_ref):
    g = g_ref[...]
    h = 0.5 * g * (1.0 + lax.erf(g * (1.0 / math.sqrt(2.0))))
    w = w_ref[...]
    c4 = jnp.concatenate([w, w, w, w], axis=0)
    wrep = jnp.concatenate([c4, c4, c4, c4], axis=1)
    rowq = lax.broadcasted_iota(jnp.int32, (_D, _PACK * _D), 0) // _D4
    colq = lax.broadcasted_iota(jnp.int32, (_D, _PACK * _D), 1) // _D
    w512 = jnp.where(rowq == colq, wrep, 0.0)
    h2 = jnp.dot(h, w512, preferred_element_type=jnp.float32)

    tyd = tyd_ref[...].reshape(1, 1, _D)
    gm = gm_ref[...].reshape(1, 1, _D)
    bt = bt_ref[...].reshape(1, 1, _D)
    tb = tt_ref[...].astype(jnp.int32)
    pieces = []
    for k in range(_PACK):
        s = h2[:, k * _D:(k + 1) * _D]
        s = s.reshape(_SPB, _Q, _D)
        bitk = ((tb >> k) & 1).astype(jnp.float32).reshape(_SPB, _Q, 1)
        s = s + posb_ref[k * _Q:(k + 1) * _Q, :][None, :, :] + bitk * tyd
        mean = jnp.mean(s, axis=-1, keepdims=True)
        d = s - mean
        var = jnp.mean(d * d, axis=-1, keepdims=True)
        pieces.append((d * lax.rsqrt(var + 1e-12)) * gm + bt)
    out_ref[...] = jnp.stack(pieces, axis=1)


def _tc_compute(gathered, ttp, proj_W, posb, tyd, gamma, beta):
    grid = (_NP // _RPB,)
    full = lambda i: (0, 0)
    return pl.pallas_call(
        _tc_body,
        grid=grid,
        in_specs=[
            pl.BlockSpec((_RPB, _D), lambda i: (i, 0)),
            pl.BlockSpec((_RPB, 1), lambda i: (i, 0)),
            pl.BlockSpec((_D4, _D), full),
            pl.BlockSpec((_S, _D), full),
            pl.BlockSpec((1, _D), full),
            pl.BlockSpec((1, _D), full),
            pl.BlockSpec((1, _D), full),
        ],
        out_specs=pl.BlockSpec((_SPB, _PACK, _Q, _D), lambda i: (i, 0, 0, 0)),
        out_shape=jax.ShapeDtypeStruct((_B, _PACK, _Q, _D), jnp.float32),
    )(gathered, ttp, proj_W, posb, tyd, gamma, beta)


def kernel(x, token_type, token_table, proj_W, proj_b, pos_table, type_table,
           gamma, beta):
    gathered = _sc_gather(token_table, x)
    ttq = token_type.reshape(_B, _PACK, _Q)
    ttp = (ttq[:, 0] | (ttq[:, 1] << 1) | (ttq[:, 2] << 2)
           | (ttq[:, 3] << 3)).astype(jnp.int8).reshape(_NP, 1)
    posb = pos_table[:_S] + proj_b[None, :] + type_table[0][None, :]
    tyd = (type_table[1] - type_table[0]).reshape(1, _D)
    out = _tc_compute(
        gathered,
        ttp,
        proj_W,
        posb,
        tyd,
        gamma.reshape(1, _D),
        beta.reshape(1, _D),
    )
    return out.reshape(_B, _S, _D)
